# baseline (speedup 1.0000x reference)
import functools

import jax
import jax.numpy as jnp
from jax import lax
from jax.experimental import pallas as pl
from jax.experimental.pallas import tpu as pltpu
from jax.experimental.pallas import tpu_sc as plsc

MAX_LEN_ = 200
D_MODEL_ = 64
NC_ = 2
NS_ = 16
NW_ = NC_ * NS_
UNITS_ = MAX_LEN_ * 8
UPW_ = UNITS_ // NW_
UWORDS_ = 32 * 8 * 128


def _sc_body(pe_hbm, out_hbm, pe_v, tile0, tile1, sem0, sem1):
    wid = lax.axis_index("s") * NC_ + lax.axis_index("c")
    pltpu.sync_copy(pe_hbm, pe_v)
    tiles = (tile0, tile1)
    sems = (sem0, sem1)

    def build(tile_v, unit):
        s = unit // 8
        dg = unit % 8
        fbase = s * 64 + dg * 8
        aligned = (fbase // 16) * 16
        off = fbase - aligned
        vec16 = pe_v[pl.ds(aligned, 16)]
        dnums = lax.GatherDimensionNumbers(
            offset_dims=(), collapsed_slice_dims=(0,), start_index_map=(0,)
        )
        vecs = [
            lax.gather(
                vec16,
                jnp.full((16, 1), off + r, jnp.int32),
                dimension_numbers=dnums,
                slice_sizes=(1,),
                mode=lax.GatherScatterMode.PROMISE_IN_BOUNDS,
            )
            for r in range(8)
        ]

        def copy_body(c, _):
            base = c * 1024
            for r in range(8):
                for j in range(8):
                    tile_v[pl.ds(base + r * 128 + j * 16, 16)] = vecs[r]
            return 0

        lax.fori_loop(0, 32, copy_body, 0)

    def start_dma(tile_v, sem, unit):
        return pltpu.make_async_copy(
            tile_v, out_hbm.at[pl.ds(unit * UWORDS_, UWORDS_)], sem
        )

    def loop_body0(u, _):
        unit = wid * UPW_ + u
        buf = u % 2
        build(tiles[buf], unit)
        start_dma(tiles[buf], sems[buf], unit).start()
        return 0

    loop_body0(0, 0)
    loop_body0(1, 0)

    def steady(u, _):
        unit = wid * UPW_ + u
        prev_unit = unit - 2
        idx = u % 2

        def even():
            start_dma(tiles[0], sems[0], prev_unit).wait()
            build(tiles[0], unit)
            start_dma(tiles[0], sems[0], unit).start()
            return 0

        def odd():
            start_dma(tiles[1], sems[1], prev_unit).wait()
            build(tiles[1], unit)
            start_dma(tiles[1], sems[1], unit).start()
            return 0

        lax.cond(idx == 0, even, odd)
        return 0

    lax.fori_loop(2, UPW_, steady, 0)
    start_dma(tiles[0], sems[0], wid * UPW_ + UPW_ - 2).wait()
    start_dma(tiles[1], sems[1], wid * UPW_ + UPW_ - 1).wait()


def kernel(x, pe_weight):
    batch = x.shape[0]
    pe_flat = pe_weight.reshape(MAX_LEN_ * D_MODEL_)
    mesh = plsc.VectorSubcoreMesh(core_axis_name="c", subcore_axis_name="s")
    run = functools.partial(
        pl.kernel,
        out_type=jax.ShapeDtypeStruct((UNITS_ * UWORDS_,), jnp.float32),
        mesh=mesh,
        scratch_types=[
            pltpu.VMEM((MAX_LEN_ * D_MODEL_,), jnp.float32),
            pltpu.VMEM((UWORDS_,), jnp.float32),
            pltpu.VMEM((UWORDS_,), jnp.float32),
            pltpu.SemaphoreType.DMA,
            pltpu.SemaphoreType.DMA,
        ],
    )(_sc_body)
    out_1d = run(pe_flat)
    t3 = out_1d.reshape(UNITS_ * 32, 8, 128)
    t5 = t3.reshape(MAX_LEN_, 8, 32, 8, 128)
    t5 = jnp.transpose(t5, (2, 4, 0, 1, 3))
    return t5.reshape(batch, MAX_LEN_, D_MODEL_)


# speedup vs baseline: 1.3337x; 1.3337x over previous
import jax
import jax.numpy as jnp
from jax.experimental import pallas as pl
from jax.experimental.pallas import tpu as pltpu

MAX_LEN_ = 200
D_MODEL_ = 64
SB_ = 8


def _bcast_body(pe_ref, out_ref):
    out_ref[...] = jnp.broadcast_to(pe_ref[...][..., None], out_ref.shape)


def kernel(x, pe_weight):
    batch = x.shape[0]
    out_p = pl.pallas_call(
        _bcast_body,
        grid=(MAX_LEN_ // SB_,),
        in_specs=[pl.BlockSpec((SB_, D_MODEL_), lambda i: (i, 0))],
        out_specs=pl.BlockSpec((SB_, D_MODEL_, batch), lambda i: (i, 0, 0)),
        out_shape=jax.ShapeDtypeStruct((MAX_LEN_, D_MODEL_, batch), pe_weight.dtype),
    )(pe_weight)
    return jnp.transpose(out_p, (2, 0, 1))


# speedup vs baseline: 1.3370x; 1.0025x over previous
import jax
import jax.numpy as jnp
from jax.experimental import pallas as pl

MAX_LEN_ = 200
D_MODEL_ = 64
SB_ = 8
LB_ = 2048


def _bcast_body(pe_ref, out_ref):
    out_ref[...] = jnp.broadcast_to(pe_ref[...][..., None], out_ref.shape)


def kernel(x, pe_weight):
    batch = x.shape[0]
    out_p = pl.pallas_call(
        _bcast_body,
        grid=(MAX_LEN_ // SB_, batch // LB_),
        in_specs=[pl.BlockSpec((SB_, D_MODEL_), lambda i, j: (i, 0))],
        out_specs=pl.BlockSpec((SB_, D_MODEL_, LB_), lambda i, j: (i, 0, j)),
        out_shape=jax.ShapeDtypeStruct((MAX_LEN_, D_MODEL_, batch), pe_weight.dtype),
    )(pe_weight)
    return jnp.transpose(out_p, (2, 0, 1))
